# baseline (speedup 1.0000x reference)
import functools

import jax
import jax.numpy as jnp
import numpy as np
from jax.experimental import pallas as pl

H = 16
KVH = 8
HD = 128
D = 2048
TOP_K = 40
EPS = 1e-6
SCALE = HD ** -0.5
MININT = np.int32(-2147483648)


def _proj_body(x_ref, w_ref, nw_ref, o_ref, *, heads_blk, do_norm):
    y = jax.lax.dot_general(
        x_ref[...], w_ref[...], (((1,), (0,)), ((), ())),
        preferred_element_type=jnp.float32)
    for i in range(heads_blk):
        yi = y[:, i * HD:(i + 1) * HD]
        if do_norm:
            var = jnp.mean(yi * yi, axis=1, keepdims=True)
            yi = yi * jax.lax.rsqrt(var + EPS) * nw_ref[...]
        o_ref[i] = yi.astype(o_ref.dtype)


def _proj(x2d, w, norm_w, nheads, do_norm, bs, out_dtype):
    s = x2d.shape[0]
    n_s = s // bs
    body = functools.partial(_proj_body, heads_blk=nheads, do_norm=do_norm)
    return pl.pallas_call(
        body,
        grid=(n_s,),
        in_specs=[
            pl.BlockSpec((bs, D), lambda i: (i, 0)),
            pl.BlockSpec((D, nheads * HD), lambda i: (0, 0)),
            pl.BlockSpec((1, HD), lambda i: (0, 0)),
        ],
        out_specs=pl.BlockSpec((nheads, bs, HD), lambda i: (0, i, 0)),
        out_shape=jax.ShapeDtypeStruct((nheads, s, HD), out_dtype),
    )(x2d, w, norm_w.reshape(1, HD))


def _attn_body(q_ref, k_ref, v_ref, o_ref, *, bq):
    q = q_ref[0]
    k = k_ref[0]
    v = v_ref[0]
    s = jax.lax.dot_general(
        q, k, (((1,), (1,)), ((), ())),
        preferred_element_type=jnp.float32) * SCALE

    bits = jax.lax.bitcast_convert_type(s, jnp.int32)
    srt = jnp.where(bits < 0,
                    jnp.bitwise_xor(jnp.bitwise_not(bits), MININT),
                    bits)
    srt16 = (srt >> 16).astype(jnp.int16)

    def bis_hi(_, carry):
        lo, hi = carry
        mid = (lo + hi) >> 1
        m16 = (srt16 >= mid.astype(jnp.int16)).astype(jnp.int16)
        cnt = jnp.sum(m16, axis=1, keepdims=True).astype(jnp.int32)
        ok = cnt >= TOP_K
        return jnp.where(ok, mid, lo), jnp.where(ok, hi, mid)

    lo0 = jnp.full((bq, 1), -32768, jnp.int32)
    hi0 = jnp.full((bq, 1), 32767, jnp.int32)
    hstar, _ = jax.lax.fori_loop(0, 16, bis_hi, (lo0, hi0))

    hs16 = hstar.astype(jnp.int16)
    c_above = jnp.sum((srt16 > hs16).astype(jnp.int16), axis=1,
                      keepdims=True).astype(jnp.int32)
    kprime = TOP_K - c_above
    lo16m = jnp.bitwise_xor(jnp.bitwise_and(srt, 0xFFFF), 0x8000)
    vi16 = jnp.where(srt16 == hs16, lo16m.astype(jnp.int16),
                     jnp.int16(-32768))

    def bis_lo(_, carry):
        lo, hi = carry
        mid = (lo + hi) >> 1
        m16 = (vi16 >= mid.astype(jnp.int16)).astype(jnp.int16)
        cnt = jnp.sum(m16, axis=1, keepdims=True).astype(jnp.int32)
        ok = cnt >= kprime
        return jnp.where(ok, mid, lo), jnp.where(ok, hi, mid)

    lo0b = jnp.full((bq, 1), -32768, jnp.int32)
    hi0b = jnp.full((bq, 1), 32768, jnp.int32)
    mlo, _ = jax.lax.fori_loop(0, 16, bis_lo, (lo0b, hi0b))
    mstar = mlo + 32768

    t = jnp.bitwise_or(hstar << 16, mstar)
    keep = srt >= t
    m = jnp.max(s, axis=1, keepdims=True)
    p = jnp.where(keep, jnp.exp(s - m), 0.0)
    denom = jnp.sum(p, axis=1, keepdims=True)
    out = jax.lax.dot_general(
        p.astype(v.dtype), v, (((1,), (0,)), ((), ())),
        preferred_element_type=jnp.float32)
    o_ref[0] = (out / denom).astype(o_ref.dtype)


def _attn(q, k, v, bq, out_dtype):
    nh, s, hd = q.shape
    groups = nh // k.shape[0]
    n_s = s // bq
    body = functools.partial(_attn_body, bq=bq)
    return pl.pallas_call(
        body,
        grid=(nh, n_s),
        in_specs=[
            pl.BlockSpec((1, bq, hd), lambda h, i: (h, i, 0)),
            pl.BlockSpec((1, s, hd), lambda h, i: (h // groups, 0, 0)),
            pl.BlockSpec((1, s, hd), lambda h, i: (h // groups, 0, 0)),
        ],
        out_specs=pl.BlockSpec((1, bq, hd), lambda h, i: (h, i, 0)),
        out_shape=jax.ShapeDtypeStruct((nh, s, hd), out_dtype),
    )(q, k, v)


def _outproj_body(a_ref, w_ref, o_ref):
    acc = jax.lax.dot_general(
        a_ref[0], w_ref[0:HD, :], (((1,), (0,)), ((), ())),
        preferred_element_type=jnp.float32)
    for h in range(1, H):
        acc = acc + jax.lax.dot_general(
            a_ref[h], w_ref[h * HD:(h + 1) * HD, :], (((1,), (0,)), ((), ())),
            preferred_element_type=jnp.float32)
    o_ref[...] = acc


def _outproj(a, wo, bs):
    nh, s, hd = a.shape
    n_s = s // bs
    return pl.pallas_call(
        _outproj_body,
        grid=(n_s,),
        in_specs=[
            pl.BlockSpec((nh, bs, hd), lambda i: (0, i, 0)),
            pl.BlockSpec((nh * hd, D), lambda i: (0, 0)),
        ],
        out_specs=pl.BlockSpec((bs, D), lambda i: (i, 0)),
        out_shape=jax.ShapeDtypeStruct((s, D), jnp.float32),
    )(a, wo)


def kernel(x, Wq, Wk, Wv, Wo, q_norm_w, k_norm_w):
    b, s, _ = x.shape
    x2d = x.reshape(b * s, D)
    bs = 256
    q = _proj(x2d, Wq, q_norm_w, H, True, bs, jnp.float32)
    k = _proj(x2d, Wk, k_norm_w, KVH, True, bs, jnp.float32)
    v = _proj(x2d, Wv, k_norm_w, KVH, False, bs, jnp.float32)
    o = _attn(q, k, v, bs, jnp.float32)
    out = _outproj(o, Wo, bs)
    return out.reshape(b, s, D)


# speedup vs baseline: 1.2031x; 1.2031x over previous
import functools

import jax
import jax.numpy as jnp
import numpy as np
from jax.experimental import pallas as pl

H = 16
KVH = 8
HD = 128
D = 2048
TOP_K = 40
EPS = 1e-6
SCALE = HD ** -0.5
MININT = np.int32(-2147483648)


def _proj_body(x_ref, w_ref, nw_ref, o_ref, *, heads_blk, do_norm):
    y = jax.lax.dot_general(
        x_ref[...], w_ref[...], (((1,), (0,)), ((), ())),
        preferred_element_type=jnp.float32)
    for i in range(heads_blk):
        yi = y[:, i * HD:(i + 1) * HD]
        if do_norm:
            var = jnp.mean(yi * yi, axis=1, keepdims=True)
            yi = yi * jax.lax.rsqrt(var + EPS) * nw_ref[...]
        o_ref[i] = yi.astype(o_ref.dtype)


def _proj(x2d, w, norm_w, nheads, do_norm, bs, out_dtype):
    s = x2d.shape[0]
    n_s = s // bs
    body = functools.partial(_proj_body, heads_blk=nheads, do_norm=do_norm)
    return pl.pallas_call(
        body,
        grid=(n_s,),
        in_specs=[
            pl.BlockSpec((bs, D), lambda i: (i, 0)),
            pl.BlockSpec((D, nheads * HD), lambda i: (0, 0)),
            pl.BlockSpec((1, HD), lambda i: (0, 0)),
        ],
        out_specs=pl.BlockSpec((nheads, bs, HD), lambda i: (0, i, 0)),
        out_shape=jax.ShapeDtypeStruct((nheads, s, HD), out_dtype),
    )(x2d, w, norm_w.reshape(1, HD))


def _attn_body(q_ref, k_ref, v_ref, o_ref, *, bq):
    q = q_ref[0]
    k = k_ref[0]
    v = v_ref[0]
    s = jax.lax.dot_general(
        q, k, (((1,), (1,)), ((), ())),
        preferred_element_type=jnp.float32) * SCALE

    bits = jax.lax.bitcast_convert_type(s, jnp.int32)
    srt = jnp.where(bits < 0,
                    jnp.bitwise_xor(jnp.bitwise_not(bits), MININT),
                    bits)
    srt16 = (srt >> 16).astype(jnp.int16)

    def bis_hi(_, carry):
        lo, hi = carry
        mid = (lo + hi) >> 1
        m16 = (srt16 >= mid.astype(jnp.int16)).astype(jnp.int16)
        cnt = jnp.sum(m16, axis=1, keepdims=True).astype(jnp.int32)
        ok = cnt >= TOP_K
        return jnp.where(ok, mid, lo), jnp.where(ok, hi, mid)

    lo0 = jnp.full((bq, 1), -32768, jnp.int32)
    hi0 = jnp.full((bq, 1), 32767, jnp.int32)
    hstar, _ = jax.lax.fori_loop(0, 16, bis_hi, (lo0, hi0))

    hi32 = srt16.astype(jnp.int32)
    c_above = jnp.sum((hi32 > hstar).astype(jnp.int32), axis=1, keepdims=True)
    kprime = TOP_K - c_above
    lo16 = jnp.bitwise_and(srt, 0xFFFF)
    vi = jnp.where(hi32 == hstar, lo16, -1)

    def bis_lo(_, carry):
        lo, hi = carry
        mid = (lo + hi) >> 1
        cnt = jnp.sum((vi >= mid).astype(jnp.int32), axis=1, keepdims=True)
        ok = cnt >= kprime
        return jnp.where(ok, mid, lo), jnp.where(ok, hi, mid)

    lo0b = jnp.zeros((bq, 1), jnp.int32)
    hi0b = jnp.full((bq, 1), 65536, jnp.int32)
    mstar, _ = jax.lax.fori_loop(0, 17, bis_lo, (lo0b, hi0b))

    t = jnp.bitwise_or(hstar << 16, mstar)
    keep = srt >= t
    m = jnp.max(s, axis=1, keepdims=True)
    p = jnp.where(keep, jnp.exp(s - m), 0.0)
    denom = jnp.sum(p, axis=1, keepdims=True)
    out = jax.lax.dot_general(
        p.astype(v.dtype), v, (((1,), (0,)), ((), ())),
        preferred_element_type=jnp.float32)
    o_ref[0] = (out / denom).astype(o_ref.dtype)


def _attn(q, k, v, bq, out_dtype):
    nh, s, hd = q.shape
    groups = nh // k.shape[0]
    n_s = s // bq
    body = functools.partial(_attn_body, bq=bq)
    return pl.pallas_call(
        body,
        grid=(nh, n_s),
        in_specs=[
            pl.BlockSpec((1, bq, hd), lambda h, i: (h, i, 0)),
            pl.BlockSpec((1, s, hd), lambda h, i: (h // groups, 0, 0)),
            pl.BlockSpec((1, s, hd), lambda h, i: (h // groups, 0, 0)),
        ],
        out_specs=pl.BlockSpec((1, bq, hd), lambda h, i: (h, i, 0)),
        out_shape=jax.ShapeDtypeStruct((nh, s, hd), out_dtype),
    )(q, k, v)


def _outproj_body(a_ref, w_ref, o_ref):
    acc = jax.lax.dot_general(
        a_ref[0], w_ref[0:HD, :], (((1,), (0,)), ((), ())),
        preferred_element_type=jnp.float32)
    for h in range(1, H):
        acc = acc + jax.lax.dot_general(
            a_ref[h], w_ref[h * HD:(h + 1) * HD, :], (((1,), (0,)), ((), ())),
            preferred_element_type=jnp.float32)
    o_ref[...] = acc


def _outproj(a, wo, bs):
    nh, s, hd = a.shape
    n_s = s // bs
    return pl.pallas_call(
        _outproj_body,
        grid=(n_s,),
        in_specs=[
            pl.BlockSpec((nh, bs, hd), lambda i: (0, i, 0)),
            pl.BlockSpec((nh * hd, D), lambda i: (0, 0)),
        ],
        out_specs=pl.BlockSpec((bs, D), lambda i: (i, 0)),
        out_shape=jax.ShapeDtypeStruct((s, D), jnp.float32),
    )(a, wo)


def kernel(x, Wq, Wk, Wv, Wo, q_norm_w, k_norm_w):
    b, s, _ = x.shape
    x2d = x.reshape(b * s, D)
    bs = 256
    q = _proj(x2d, Wq, q_norm_w, H, True, bs, jnp.float32)
    k = _proj(x2d, Wk, k_norm_w, KVH, True, bs, jnp.float32)
    v = _proj(x2d, Wv, k_norm_w, KVH, False, bs, jnp.bfloat16)
    o = _attn(q, k, v, bs, jnp.bfloat16)
    out = _outproj(o, Wo.astype(jnp.bfloat16), bs)
    return out.reshape(b, s, D)


# speedup vs baseline: 1.2518x; 1.0405x over previous
import functools

import jax
import jax.numpy as jnp
import numpy as np
from jax.experimental import pallas as pl

H = 16
KVH = 8
HD = 128
D = 2048
TOP_K = 40
EPS = 1e-6
SCALE = HD ** -0.5
MININT = np.int32(-2147483648)


def _proj_body(x_ref, w_ref, nw_ref, o_ref, *, heads_blk, do_norm):
    y = jax.lax.dot_general(
        x_ref[...], w_ref[...], (((1,), (0,)), ((), ())),
        preferred_element_type=jnp.float32)
    for i in range(heads_blk):
        yi = y[:, i * HD:(i + 1) * HD]
        if do_norm:
            var = jnp.mean(yi * yi, axis=1, keepdims=True)
            yi = yi * jax.lax.rsqrt(var + EPS) * nw_ref[...]
        o_ref[i] = yi.astype(o_ref.dtype)


def _proj(x2d, w, norm_w, nheads, do_norm, bs, out_dtype):
    s = x2d.shape[0]
    n_s = s // bs
    body = functools.partial(_proj_body, heads_blk=nheads, do_norm=do_norm)
    return pl.pallas_call(
        body,
        grid=(n_s,),
        in_specs=[
            pl.BlockSpec((bs, D), lambda i: (i, 0)),
            pl.BlockSpec((D, nheads * HD), lambda i: (0, 0)),
            pl.BlockSpec((1, HD), lambda i: (0, 0)),
        ],
        out_specs=pl.BlockSpec((nheads, bs, HD), lambda i: (0, i, 0)),
        out_shape=jax.ShapeDtypeStruct((nheads, s, HD), out_dtype),
    )(x2d, w, norm_w.reshape(1, HD))


def _attn_body(q_ref, k_ref, v_ref, o_ref, *, bq):
    q = q_ref[0]
    k = k_ref[0]
    v = v_ref[0]
    s = jax.lax.dot_general(
        q, k, (((1,), (1,)), ((), ())),
        preferred_element_type=jnp.float32) * SCALE

    bits = jax.lax.bitcast_convert_type(s, jnp.int32)
    srt = jnp.where(bits < 0,
                    jnp.bitwise_xor(jnp.bitwise_not(bits), MININT),
                    bits)
    srt16 = (srt >> 16).astype(jnp.int16)

    def bis_hi(_, carry):
        lo, hi = carry
        mid = (lo + hi) >> 1
        m16 = (srt16 >= mid.astype(jnp.int16)).astype(jnp.int16)
        cnt = jnp.sum(m16, axis=1, keepdims=True).astype(jnp.int32)
        ok = cnt >= TOP_K
        return jnp.where(ok, mid, lo), jnp.where(ok, hi, mid)

    lo0 = jnp.full((bq, 1), -32768, jnp.int32)
    hi0 = jnp.full((bq, 1), 32767, jnp.int32)
    hstar, _ = jax.lax.fori_loop(0, 16, bis_hi, (lo0, hi0))

    hi32 = srt16.astype(jnp.int32)
    c_above = jnp.sum((hi32 > hstar).astype(jnp.int32), axis=1, keepdims=True)
    kprime = TOP_K - c_above
    lo16 = jnp.bitwise_and(srt, 0xFFFF)
    vi = jnp.where(hi32 == hstar, lo16, -1)

    def bis_lo(_, carry):
        lo, hi = carry
        mid = (lo + hi) >> 1
        cnt = jnp.sum((vi >= mid).astype(jnp.int32), axis=1, keepdims=True)
        ok = cnt >= kprime
        return jnp.where(ok, mid, lo), jnp.where(ok, hi, mid)

    lo0b = jnp.zeros((bq, 1), jnp.int32)
    hi0b = jnp.full((bq, 1), 65536, jnp.int32)
    mstar, _ = jax.lax.fori_loop(0, 17, bis_lo, (lo0b, hi0b))

    t = jnp.bitwise_or(hstar << 16, mstar)
    keep = srt >= t
    m = jnp.max(s, axis=1, keepdims=True)
    p = jnp.where(keep, jnp.exp(s - m), 0.0)
    denom = jnp.sum(p, axis=1, keepdims=True)
    out = jax.lax.dot_general(
        p.astype(v.dtype), v, (((1,), (0,)), ((), ())),
        preferred_element_type=jnp.float32)
    o_ref[0] = (out / denom).astype(o_ref.dtype)


def _attn(q, k, v, bq, out_dtype):
    nh, s, hd = q.shape
    groups = nh // k.shape[0]
    n_s = s // bq
    body = functools.partial(_attn_body, bq=bq)
    return pl.pallas_call(
        body,
        grid=(nh, n_s),
        in_specs=[
            pl.BlockSpec((1, bq, hd), lambda h, i: (h, i, 0)),
            pl.BlockSpec((1, s, hd), lambda h, i: (h // groups, 0, 0)),
            pl.BlockSpec((1, s, hd), lambda h, i: (h // groups, 0, 0)),
        ],
        out_specs=pl.BlockSpec((1, bq, hd), lambda h, i: (h, i, 0)),
        out_shape=jax.ShapeDtypeStruct((nh, s, hd), out_dtype),
    )(q, k, v)


def _outproj_body(a_ref, w_ref, o_ref):
    acc = jax.lax.dot_general(
        a_ref[0], w_ref[0:HD, :], (((1,), (0,)), ((), ())),
        preferred_element_type=jnp.float32)
    for h in range(1, H):
        acc = acc + jax.lax.dot_general(
            a_ref[h], w_ref[h * HD:(h + 1) * HD, :], (((1,), (0,)), ((), ())),
            preferred_element_type=jnp.float32)
    o_ref[...] = acc


def _outproj(a, wo, bs):
    nh, s, hd = a.shape
    n_s = s // bs
    return pl.pallas_call(
        _outproj_body,
        grid=(n_s,),
        in_specs=[
            pl.BlockSpec((nh, bs, hd), lambda i: (0, i, 0)),
            pl.BlockSpec((nh * hd, D), lambda i: (0, 0)),
        ],
        out_specs=pl.BlockSpec((bs, D), lambda i: (i, 0)),
        out_shape=jax.ShapeDtypeStruct((s, D), jnp.float32),
    )(a, wo)


def kernel(x, Wq, Wk, Wv, Wo, q_norm_w, k_norm_w):
    b, s, _ = x.shape
    x2d = x.reshape(b * s, D)
    bs = 256
    q = _proj(x2d, Wq, q_norm_w, H, True, bs, jnp.float32)
    k = _proj(x2d, Wk, k_norm_w, KVH, True, bs, jnp.float32)
    v = _proj(x2d, Wv, k_norm_w, KVH, False, bs, jnp.bfloat16)
    o = _attn(q, k, v, 512, jnp.bfloat16)
    out = _outproj(o, Wo.astype(jnp.bfloat16), bs)
    return out.reshape(b, s, D)


# speedup vs baseline: 1.2787x; 1.0215x over previous
import functools

import jax
import jax.numpy as jnp
import numpy as np
from jax.experimental import pallas as pl

H = 16
KVH = 8
HD = 128
D = 2048
TOP_K = 40
EPS = 1e-6
SCALE = HD ** -0.5
MININT = np.int32(-2147483648)


def _proj_body(x_ref, w_ref, nw_ref, o_ref, *, heads_blk, do_norm):
    y = jax.lax.dot_general(
        x_ref[...], w_ref[...], (((1,), (0,)), ((), ())),
        preferred_element_type=jnp.float32)
    for i in range(heads_blk):
        yi = y[:, i * HD:(i + 1) * HD]
        if do_norm:
            var = jnp.mean(yi * yi, axis=1, keepdims=True)
            yi = yi * jax.lax.rsqrt(var + EPS) * nw_ref[...]
        o_ref[i] = yi.astype(o_ref.dtype)


def _proj(x2d, w, norm_w, nheads, do_norm, bs, out_dtype):
    s = x2d.shape[0]
    n_s = s // bs
    body = functools.partial(_proj_body, heads_blk=nheads, do_norm=do_norm)
    return pl.pallas_call(
        body,
        grid=(n_s,),
        in_specs=[
            pl.BlockSpec((bs, D), lambda i: (i, 0)),
            pl.BlockSpec((D, nheads * HD), lambda i: (0, 0)),
            pl.BlockSpec((1, HD), lambda i: (0, 0)),
        ],
        out_specs=pl.BlockSpec((nheads, bs, HD), lambda i: (0, i, 0)),
        out_shape=jax.ShapeDtypeStruct((nheads, s, HD), out_dtype),
    )(x2d, w, norm_w.reshape(1, HD))


def _attn_body(q_ref, k_ref, v_ref, o_ref, *, bq):
    q = q_ref[0]
    k = k_ref[0]
    v = v_ref[0]
    s = jax.lax.dot_general(
        q, k, (((1,), (1,)), ((), ())),
        preferred_element_type=jnp.float32) * SCALE

    bits = jax.lax.bitcast_convert_type(s, jnp.int32)
    srt = jnp.where(bits < 0,
                    jnp.bitwise_xor(jnp.bitwise_not(bits), MININT),
                    bits)
    srt16 = (srt >> 16).astype(jnp.int16)

    def bis_hi(_, carry):
        lo, hi = carry
        mid = (lo + hi) >> 1
        m16 = (srt16 >= mid.astype(jnp.int16)).astype(jnp.int16)
        cnt = jnp.sum(m16, axis=1, keepdims=True).astype(jnp.int32)
        ok = cnt >= TOP_K
        return jnp.where(ok, mid, lo), jnp.where(ok, hi, mid)

    lo0 = jnp.full((bq, 1), -32768, jnp.int32)
    hi0 = jnp.full((bq, 1), 32767, jnp.int32)
    hstar, _ = jax.lax.fori_loop(0, 16, bis_hi, (lo0, hi0))

    hi32 = srt16.astype(jnp.int32)
    c_above = jnp.sum((hi32 > hstar).astype(jnp.int32), axis=1, keepdims=True)
    kprime = TOP_K - c_above
    lo16 = jnp.bitwise_and(srt, 0xFFFF)
    vi = jnp.where(hi32 == hstar, lo16, -1)

    def bis_lo(_, carry):
        lo, hi = carry
        mid = (lo + hi) >> 1
        cnt = jnp.sum((vi >= mid).astype(jnp.int32), axis=1, keepdims=True)
        ok = cnt >= kprime
        return jnp.where(ok, mid, lo), jnp.where(ok, hi, mid)

    lo0b = jnp.zeros((bq, 1), jnp.int32)
    hi0b = jnp.full((bq, 1), 65536, jnp.int32)
    mstar, _ = jax.lax.fori_loop(0, 17, bis_lo, (lo0b, hi0b))

    t = jnp.bitwise_or(hstar << 16, mstar)
    keep = srt >= t
    m = jnp.max(s, axis=1, keepdims=True)
    p = jnp.where(keep, jnp.exp(s - m), 0.0)
    denom = jnp.sum(p, axis=1, keepdims=True)
    out = jax.lax.dot_general(
        p.astype(v.dtype), v, (((1,), (0,)), ((), ())),
        preferred_element_type=jnp.float32)
    o_ref[0] = (out / denom).astype(o_ref.dtype)


def _attn(q, k, v, bq, out_dtype):
    nh, s, hd = q.shape
    groups = nh // k.shape[0]
    n_s = s // bq
    body = functools.partial(_attn_body, bq=bq)
    return pl.pallas_call(
        body,
        grid=(nh, n_s),
        in_specs=[
            pl.BlockSpec((1, bq, hd), lambda h, i: (h, i, 0)),
            pl.BlockSpec((1, s, hd), lambda h, i: (h // groups, 0, 0)),
            pl.BlockSpec((1, s, hd), lambda h, i: (h // groups, 0, 0)),
        ],
        out_specs=pl.BlockSpec((1, bq, hd), lambda h, i: (h, i, 0)),
        out_shape=jax.ShapeDtypeStruct((nh, s, hd), out_dtype),
    )(q, k, v)


def _outproj_body(a_ref, w_ref, o_ref):
    acc = jax.lax.dot_general(
        a_ref[0], w_ref[0:HD, :], (((1,), (0,)), ((), ())),
        preferred_element_type=jnp.float32)
    for h in range(1, H):
        acc = acc + jax.lax.dot_general(
            a_ref[h], w_ref[h * HD:(h + 1) * HD, :], (((1,), (0,)), ((), ())),
            preferred_element_type=jnp.float32)
    o_ref[...] = acc


def _outproj(a, wo, bs):
    nh, s, hd = a.shape
    n_s = s // bs
    return pl.pallas_call(
        _outproj_body,
        grid=(n_s,),
        in_specs=[
            pl.BlockSpec((nh, bs, hd), lambda i: (0, i, 0)),
            pl.BlockSpec((nh * hd, D), lambda i: (0, 0)),
        ],
        out_specs=pl.BlockSpec((bs, D), lambda i: (i, 0)),
        out_shape=jax.ShapeDtypeStruct((s, D), jnp.float32),
    )(a, wo)


def kernel(x, Wq, Wk, Wv, Wo, q_norm_w, k_norm_w):
    b, s, _ = x.shape
    x2d = x.reshape(b * s, D)
    bs = 256
    q = _proj(x2d, Wq, q_norm_w, H, True, bs, jnp.float32)
    k = _proj(x2d, Wk, k_norm_w, KVH, True, bs, jnp.float32)
    v = _proj(x2d, Wv, k_norm_w, KVH, False, bs, jnp.bfloat16)
    o = _attn(q, k, v, 1024, jnp.bfloat16)
    out = _outproj(o, Wo.astype(jnp.bfloat16), bs)
    return out.reshape(b, s, D)
